# Initial kernel scaffold; baseline (speedup 1.0000x reference)
import jax
import jax.numpy as jnp
from jax.experimental import pallas as pl


def kernel(x, edge_index, W1_l, b1_l, W1_r, W2_l, b2_l, W2_r):
    raise NotImplementedError("write your pallas kernel here")


# speedup vs baseline: 6.3906x; 6.3906x over previous
import functools

import jax
import jax.numpy as jnp
from jax import lax
from jax.experimental import pallas as pl
from jax.experimental.pallas import tpu as pltpu
from jax.experimental.pallas import tpu_sc as plsc

N = 10000
E = 320000
D = 128
DEGW = 16
DAUG = D + DEGW
NC, NS = 2, 16
NW = NC * NS
EPW = E // NW
CHUNK = 128
NFULL = EPW // CHUNK
REM = EPW - NFULL * CHUNK
RPSA = 624
NTAIL = N - NS * RPSA
NBASE = NS * RPSA


def _sc_agg_body(width, *refs):
    (x_hbm, src_hbm, dst_hbm, z_hbm,
     out_hbm,
     idx_s, idx_d, idx_s16, idx_d16, idxz, idxz112, rows, rows16,
     acc, sem) = refs

    cid = lax.axis_index("c")
    sid = lax.axis_index("s")
    wid = cid * NS + sid
    r0 = pl.multiple_of(sid * RPSA, 16)
    _OFFS = ((0, 128), (128, 128), (256, 128), (384, 128), (512, 112))

    def fill_idx(ref, base, n):
        lanes = lax.iota(jnp.int32, 16)
        for j in range(0, n, 16):
            ref[pl.ds(j, 16)] = lanes + (base + j)

    pltpu.sync_copy(z_hbm, rows)
    for o, s in _OFFS:
        iref = idxz if s == 128 else idxz112
        fill_idx(iref, r0 + o, s)
        pltpu.sync_copy(rows.at[pl.ds(0, s)], acc.at[iref])

    @pl.when(sid == NS - 1)
    def _zero_tail():
        fill_idx(idx_s16, NBASE, NTAIL)
        pltpu.sync_copy(rows.at[pl.ds(0, NTAIL)], acc.at[idx_s16])

    plsc.subcore_barrier()

    ebase = wid * EPW

    def chunk_body(i, carry):
        base = ebase + i * CHUNK
        pltpu.sync_copy(src_hbm.at[pl.ds(base, CHUNK)], idx_s)
        pltpu.sync_copy(dst_hbm.at[pl.ds(base, CHUNK)], idx_d)
        pltpu.async_copy(x_hbm.at[idx_s], rows, sem).wait()
        pltpu.sync_copy(rows, acc.at[idx_d], add=True)
        return carry

    lax.fori_loop(0, NFULL, chunk_body, 0)

    base = ebase + NFULL * CHUNK
    pltpu.sync_copy(src_hbm.at[pl.ds(base, REM)], idx_s16)
    pltpu.sync_copy(dst_hbm.at[pl.ds(base, REM)], idx_d16)
    pltpu.async_copy(x_hbm.at[idx_s16], rows16, sem).wait()
    pltpu.sync_copy(rows16, acc.at[idx_d16], add=True)

    plsc.subcore_barrier()

    for o, s in _OFFS:
        iref = idxz if s == 128 else idxz112
        fill_idx(iref, r0 + o, s)
        pltpu.async_copy(acc.at[iref], rows.at[pl.ds(0, s)], sem).wait()
        pltpu.sync_copy(rows.at[pl.ds(0, s)],
                        out_hbm.at[cid, pl.ds(r0 + o, s)])

    @pl.when(sid == NS - 1)
    def _wb_tail():
        fill_idx(idx_s16, NBASE, NTAIL)
        pltpu.async_copy(acc.at[idx_s16], rows16, sem).wait()
        pltpu.sync_copy(rows16, out_hbm.at[cid, pl.ds(NBASE, NTAIL)])


@functools.lru_cache(maxsize=None)
def _build_sc_kernels():
    mesh = plsc.VectorSubcoreMesh(
        core_axis_name="c", subcore_axis_name="s",
        num_cores=NC, num_subcores=NS)

    def make(width):
        return pl.kernel(
            functools.partial(_sc_agg_body, width),
            out_type=jax.ShapeDtypeStruct((NC, N, width), jnp.float32),
            mesh=mesh,
            compiler_params=pltpu.CompilerParams(use_tc_tiling_on_sc=False),
            scratch_types=[
                pltpu.VMEM((CHUNK,), jnp.int32),
                pltpu.VMEM((CHUNK,), jnp.int32),
                pltpu.VMEM((REM,), jnp.int32),
                pltpu.VMEM((REM,), jnp.int32),
                pltpu.VMEM((CHUNK,), jnp.int32),
                pltpu.VMEM((112,), jnp.int32),
                pltpu.VMEM((CHUNK, width), jnp.float32),
                pltpu.VMEM((REM, width), jnp.float32),
                pltpu.VMEM_SHARED((N, width), jnp.float32),
                pltpu.SemaphoreType.DMA,
            ],
        )

    return make(DAUG), make(D)


_R = 1000


def _dense1_body(p_ref, x_ref, wl_ref, wr_ref, b_ref, o_ref):
    p = p_ref[0] + p_ref[1]
    s = jnp.sum(p[:, D:], axis=1)
    inv = 1.0 / jnp.maximum(s * (1.0 / DEGW), 1.0)
    mean = p[:, :D] * inv[:, None]
    z = (jnp.dot(mean, wl_ref[...], preferred_element_type=jnp.float32)
         + jnp.dot(x_ref[...], wr_ref[...], preferred_element_type=jnp.float32)
         + b_ref[...])
    o_ref[...] = jnp.maximum(z, 0.0)


def _dense2_body(p_ref, deg_ref, h_ref, wl_ref, wr_ref, b_ref, o_ref):
    s = jnp.sum(deg_ref[0] + deg_ref[1], axis=1)
    inv = 1.0 / jnp.maximum(s * (1.0 / DEGW), 1.0)
    mean = (p_ref[0] + p_ref[1]) * inv[:, None]
    z = (jnp.dot(mean, wl_ref[...], preferred_element_type=jnp.float32)
         + jnp.dot(h_ref[...], wr_ref[...], preferred_element_type=jnp.float32)
         + b_ref[...])
    m = jnp.max(z, axis=1, keepdims=True)
    lse = jnp.log(jnp.sum(jnp.exp(z - m), axis=1, keepdims=True)) + m
    o_ref[...] = z - lse


_dense1 = pl.pallas_call(
    _dense1_body,
    grid=(N // _R,),
    in_specs=[
        pl.BlockSpec((NC, _R, DAUG), lambda i: (0, i, 0)),
        pl.BlockSpec((_R, D), lambda i: (i, 0)),
        pl.BlockSpec((D, D), lambda i: (0, 0)),
        pl.BlockSpec((D, D), lambda i: (0, 0)),
        pl.BlockSpec((1, D), lambda i: (0, 0)),
    ],
    out_specs=pl.BlockSpec((_R, D), lambda i: (i, 0)),
    out_shape=jax.ShapeDtypeStruct((N, D), jnp.float32),
)

_dense2 = pl.pallas_call(
    _dense2_body,
    grid=(N // _R,),
    in_specs=[
        pl.BlockSpec((NC, _R, D), lambda i: (0, i, 0)),
        pl.BlockSpec((NC, _R, DEGW), lambda i: (0, i, 0)),
        pl.BlockSpec((_R, D), lambda i: (i, 0)),
        pl.BlockSpec((D, D), lambda i: (0, 0)),
        pl.BlockSpec((D, D), lambda i: (0, 0)),
        pl.BlockSpec((1, D), lambda i: (0, 0)),
    ],
    out_specs=pl.BlockSpec((_R, D), lambda i: (i, 0)),
    out_shape=jax.ShapeDtypeStruct((N, D), jnp.float32),
)


def kernel(x, edge_index, W1_l, b1_l, W1_r, W2_l, b2_l, W2_r):
    src = edge_index[0].astype(jnp.int32)
    dst = edge_index[1].astype(jnp.int32)
    x_aug = jnp.concatenate([x, jnp.ones((N, DEGW), jnp.float32)], axis=1)
    z144 = jnp.zeros((CHUNK, DAUG), jnp.float32)
    z128 = jnp.zeros((CHUNK, D), jnp.float32)

    agg_aug, agg = _build_sc_kernels()
    p1 = agg_aug(x_aug, src, dst, z144)
    h = _dense1(p1, x, W1_l.T, W1_r.T, b1_l.reshape(1, D))
    p2 = agg(h, src, dst, z128)
    degc = lax.slice_in_dim(p1, D, DAUG, axis=2)
    out = _dense2(p2, degc, h, W2_l.T, W2_r.T, b2_l.reshape(1, D))
    return out


# speedup vs baseline: 7.2832x; 1.1397x over previous
import functools

import jax
import jax.numpy as jnp
from jax import lax
from jax.experimental import pallas as pl
from jax.experimental.pallas import tpu as pltpu
from jax.experimental.pallas import tpu_sc as plsc

N = 10000
E = 320000
D = 128
DEGW = 16
DAUG = D + DEGW
NC, NS = 2, 16
NW = NC * NS
EPW = E // NW
CHUNK = 128
NFULL = EPW // CHUNK
REM = EPW - NFULL * CHUNK
RPSA = 624
NTAIL = N - NS * RPSA
NBASE = NS * RPSA


def _sc_agg_body(width, *refs):
    (x_hbm, src_hbm, dst_hbm, z_hbm,
     out_hbm,
     idx_sab, idx_d_a, idx_d_b, idx_s16, idx_d16, idxz, idxz112,
     rows, rows_b, rows16,
     acc, sem, sem_b, sem_s) = refs

    cid = lax.axis_index("c")
    sid = lax.axis_index("s")
    wid = cid * NS + sid
    r0 = pl.multiple_of(sid * RPSA, 16)
    _OFFS = ((0, 128), (128, 128), (256, 128), (384, 128), (512, 112))

    def fill_idx(ref, base, n):
        lanes = lax.iota(jnp.int32, 16)
        for j in range(0, n, 16):
            ref[pl.ds(j, 16)] = lanes + (base + j)

    pltpu.sync_copy(z_hbm, rows)
    for o, s in _OFFS:
        iref = idxz if s == 128 else idxz112
        fill_idx(iref, r0 + o, s)
        pltpu.sync_copy(rows.at[pl.ds(0, s)], acc.at[iref])

    @pl.when(sid == NS - 1)
    def _zero_tail():
        fill_idx(idx_s16, NBASE, NTAIL)
        pltpu.sync_copy(rows.at[pl.ds(0, NTAIL)], acc.at[idx_s16])

    plsc.subcore_barrier()

    ebase = wid * EPW

    def chunk_body(i, carry):
        base = ebase + i * (2 * CHUNK)
        pltpu.sync_copy(src_hbm.at[pl.ds(base, 2 * CHUNK)], idx_sab)
        pltpu.sync_copy(dst_hbm.at[pl.ds(base, CHUNK)], idx_d_a)
        pltpu.sync_copy(dst_hbm.at[pl.ds(base + CHUNK, CHUNK)], idx_d_b)
        ga = pltpu.async_copy(
            x_hbm.at[idx_sab.at[pl.ds(0, CHUNK)]], rows, sem)
        gb = pltpu.async_copy(
            x_hbm.at[idx_sab.at[pl.ds(CHUNK, CHUNK)]], rows_b, sem_b)
        ga.wait()
        sa = pltpu.async_copy(rows, acc.at[idx_d_a], sem_s, add=True)
        gb.wait()
        sb = pltpu.async_copy(rows_b, acc.at[idx_d_b], sem_s, add=True)
        sa.wait()
        sb.wait()
        return carry

    lax.fori_loop(0, NFULL // 2, chunk_body, 0)

    base = ebase + NFULL * CHUNK
    pltpu.sync_copy(src_hbm.at[pl.ds(base, REM)], idx_s16)
    pltpu.sync_copy(dst_hbm.at[pl.ds(base, REM)], idx_d16)
    pltpu.async_copy(x_hbm.at[idx_s16], rows16, sem).wait()
    pltpu.sync_copy(rows16, acc.at[idx_d16], add=True)

    plsc.subcore_barrier()

    for o, s in _OFFS:
        iref = idxz if s == 128 else idxz112
        fill_idx(iref, r0 + o, s)
        pltpu.async_copy(acc.at[iref], rows.at[pl.ds(0, s)], sem).wait()
        pltpu.sync_copy(rows.at[pl.ds(0, s)],
                        out_hbm.at[cid, pl.ds(r0 + o, s)])

    @pl.when(sid == NS - 1)
    def _wb_tail():
        fill_idx(idx_s16, NBASE, NTAIL)
        pltpu.async_copy(acc.at[idx_s16], rows16, sem).wait()
        pltpu.sync_copy(rows16, out_hbm.at[cid, pl.ds(NBASE, NTAIL)])


@functools.lru_cache(maxsize=None)
def _build_sc_kernels():
    mesh = plsc.VectorSubcoreMesh(
        core_axis_name="c", subcore_axis_name="s",
        num_cores=NC, num_subcores=NS)

    def make(width):
        return pl.kernel(
            functools.partial(_sc_agg_body, width),
            out_type=jax.ShapeDtypeStruct((NC, N, width), jnp.float32),
            mesh=mesh,
            compiler_params=pltpu.CompilerParams(use_tc_tiling_on_sc=False),
            scratch_types=[
                pltpu.VMEM((2 * CHUNK,), jnp.int32),
                pltpu.VMEM((CHUNK,), jnp.int32),
                pltpu.VMEM((CHUNK,), jnp.int32),
                pltpu.VMEM((REM,), jnp.int32),
                pltpu.VMEM((REM,), jnp.int32),
                pltpu.VMEM((CHUNK,), jnp.int32),
                pltpu.VMEM((112,), jnp.int32),
                pltpu.VMEM((CHUNK, width), jnp.float32),
                pltpu.VMEM((CHUNK, width), jnp.float32),
                pltpu.VMEM((REM, width), jnp.float32),
                pltpu.VMEM_SHARED((N, width), jnp.float32),
                pltpu.SemaphoreType.DMA,
                pltpu.SemaphoreType.DMA,
                pltpu.SemaphoreType.DMA,
            ],
        )

    return make(DAUG), make(D)


_R = 1000


def _dense1_body(p_ref, x_ref, wl_ref, wr_ref, b_ref, o_ref):
    p = p_ref[0] + p_ref[1]
    s = jnp.sum(p[:, D:], axis=1)
    inv = 1.0 / jnp.maximum(s * (1.0 / DEGW), 1.0)
    mean = p[:, :D] * inv[:, None]
    z = (jnp.dot(mean, wl_ref[...], preferred_element_type=jnp.float32)
         + jnp.dot(x_ref[...], wr_ref[...], preferred_element_type=jnp.float32)
         + b_ref[...])
    o_ref[...] = jnp.maximum(z, 0.0)


def _dense2_body(p_ref, deg_ref, h_ref, wl_ref, wr_ref, b_ref, o_ref):
    s = jnp.sum(deg_ref[0] + deg_ref[1], axis=1)
    inv = 1.0 / jnp.maximum(s * (1.0 / DEGW), 1.0)
    mean = (p_ref[0] + p_ref[1]) * inv[:, None]
    z = (jnp.dot(mean, wl_ref[...], preferred_element_type=jnp.float32)
         + jnp.dot(h_ref[...], wr_ref[...], preferred_element_type=jnp.float32)
         + b_ref[...])
    m = jnp.max(z, axis=1, keepdims=True)
    lse = jnp.log(jnp.sum(jnp.exp(z - m), axis=1, keepdims=True)) + m
    o_ref[...] = z - lse


_dense1 = pl.pallas_call(
    _dense1_body,
    grid=(N // _R,),
    in_specs=[
        pl.BlockSpec((NC, _R, DAUG), lambda i: (0, i, 0)),
        pl.BlockSpec((_R, D), lambda i: (i, 0)),
        pl.BlockSpec((D, D), lambda i: (0, 0)),
        pl.BlockSpec((D, D), lambda i: (0, 0)),
        pl.BlockSpec((1, D), lambda i: (0, 0)),
    ],
    out_specs=pl.BlockSpec((_R, D), lambda i: (i, 0)),
    out_shape=jax.ShapeDtypeStruct((N, D), jnp.float32),
)

_dense2 = pl.pallas_call(
    _dense2_body,
    grid=(N // _R,),
    in_specs=[
        pl.BlockSpec((NC, _R, D), lambda i: (0, i, 0)),
        pl.BlockSpec((NC, _R, DEGW), lambda i: (0, i, 0)),
        pl.BlockSpec((_R, D), lambda i: (i, 0)),
        pl.BlockSpec((D, D), lambda i: (0, 0)),
        pl.BlockSpec((D, D), lambda i: (0, 0)),
        pl.BlockSpec((1, D), lambda i: (0, 0)),
    ],
    out_specs=pl.BlockSpec((_R, D), lambda i: (i, 0)),
    out_shape=jax.ShapeDtypeStruct((N, D), jnp.float32),
)


def kernel(x, edge_index, W1_l, b1_l, W1_r, W2_l, b2_l, W2_r):
    src = edge_index[0].astype(jnp.int32)
    dst = edge_index[1].astype(jnp.int32)
    x_aug = jnp.concatenate([x, jnp.ones((N, DEGW), jnp.float32)], axis=1)
    z144 = jnp.zeros((CHUNK, DAUG), jnp.float32)
    z128 = jnp.zeros((CHUNK, D), jnp.float32)

    agg_aug, agg = _build_sc_kernels()
    p1 = agg_aug(x_aug, src, dst, z144)
    h = _dense1(p1, x, W1_l.T, W1_r.T, b1_l.reshape(1, D))
    p2 = agg(h, src, dst, z128)
    degc = lax.slice_in_dim(p1, D, DAUG, axis=2)
    out = _dense2(p2, degc, h, W2_l.T, W2_r.T, b2_l.reshape(1, D))
    return out


# speedup vs baseline: 8.2915x; 1.1384x over previous
import functools

import jax
import jax.numpy as jnp
from jax import lax
from jax.experimental import pallas as pl
from jax.experimental.pallas import tpu as pltpu
from jax.experimental.pallas import tpu_sc as plsc

N = 10000
E = 320000
D = 128
DEGW = 16
DAUG = D + DEGW
NC, NS = 2, 16
NW = NC * NS
EPW = E // NW
CHUNK = 128
NFULL = EPW // CHUNK
REM = EPW - NFULL * CHUNK
RPSA = 624
NTAIL = N - NS * RPSA
NBASE = NS * RPSA


def _sc_agg_body(width, *refs):
    (x_hbm, pk_hbm, rem_hbm, z_hbm,
     out_hbm,
     idxbuf, idx_r, idx16, idxz, idxz112,
     rows, rows_b, rows16,
     acc, sem, sem_b, sem_s) = refs

    cid = lax.axis_index("c")
    sid = lax.axis_index("s")
    wid = cid * NS + sid
    r0 = pl.multiple_of(sid * RPSA, 16)
    _OFFS = ((0, 128), (128, 128), (256, 128), (384, 128), (512, 112))

    def fill_idx(ref, base, n):
        lanes = lax.iota(jnp.int32, 16)
        for j in range(0, n, 16):
            ref[pl.ds(j, 16)] = lanes + (base + j)

    pltpu.sync_copy(z_hbm, rows)
    for o, s in _OFFS:
        iref = idxz if s == 128 else idxz112
        fill_idx(iref, r0 + o, s)
        pltpu.sync_copy(rows.at[pl.ds(0, s)], acc.at[iref])

    @pl.when(sid == NS - 1)
    def _zero_tail():
        fill_idx(idx16, NBASE, NTAIL)
        pltpu.sync_copy(rows.at[pl.ds(0, NTAIL)], acc.at[idx16])

    plsc.subcore_barrier()

    ebase = wid * EPW

    def chunk_body(i, carry):
        pltpu.sync_copy(pk_hbm.at[wid, i], idxbuf)
        ga = pltpu.async_copy(
            x_hbm.at[idxbuf.at[pl.ds(0, CHUNK)]], rows, sem)
        gb = pltpu.async_copy(
            x_hbm.at[idxbuf.at[pl.ds(CHUNK, CHUNK)]], rows_b, sem_b)
        ga.wait()
        sa = pltpu.async_copy(rows, acc.at[idxbuf.at[pl.ds(2 * CHUNK, CHUNK)]],
                              sem_s, add=True)
        gb.wait()
        sb = pltpu.async_copy(rows_b,
                              acc.at[idxbuf.at[pl.ds(3 * CHUNK, CHUNK)]],
                              sem_s, add=True)
        sa.wait()
        sb.wait()
        return carry

    lax.fori_loop(0, NFULL // 2, chunk_body, 0)

    pltpu.sync_copy(rem_hbm.at[wid], idx_r)
    pltpu.async_copy(x_hbm.at[idx_r.at[pl.ds(0, REM)]], rows16, sem).wait()
    pltpu.sync_copy(rows16, acc.at[idx_r.at[pl.ds(REM, REM)]], add=True)

    plsc.subcore_barrier()

    for o, s in _OFFS:
        iref = idxz if s == 128 else idxz112
        fill_idx(iref, r0 + o, s)
        pltpu.async_copy(acc.at[iref], rows.at[pl.ds(0, s)], sem).wait()
        pltpu.sync_copy(rows.at[pl.ds(0, s)],
                        out_hbm.at[cid, pl.ds(r0 + o, s)])

    @pl.when(sid == NS - 1)
    def _wb_tail():
        fill_idx(idx16, NBASE, NTAIL)
        pltpu.async_copy(acc.at[idx16], rows16, sem).wait()
        pltpu.sync_copy(rows16, out_hbm.at[cid, pl.ds(NBASE, NTAIL)])


@functools.lru_cache(maxsize=None)
def _build_sc_kernels():
    mesh = plsc.VectorSubcoreMesh(
        core_axis_name="c", subcore_axis_name="s",
        num_cores=NC, num_subcores=NS)

    def make(width):
        return pl.kernel(
            functools.partial(_sc_agg_body, width),
            out_type=jax.ShapeDtypeStruct((NC, N, width), jnp.float32),
            mesh=mesh,
            compiler_params=pltpu.CompilerParams(use_tc_tiling_on_sc=False),
            scratch_types=[
                pltpu.VMEM((4 * CHUNK,), jnp.int32),
                pltpu.VMEM((2 * REM,), jnp.int32),
                pltpu.VMEM((REM,), jnp.int32),
                pltpu.VMEM((CHUNK,), jnp.int32),
                pltpu.VMEM((112,), jnp.int32),
                pltpu.VMEM((CHUNK, width), jnp.float32),
                pltpu.VMEM((CHUNK, width), jnp.float32),
                pltpu.VMEM((REM, width), jnp.float32),
                pltpu.VMEM_SHARED((N, width), jnp.float32),
                pltpu.SemaphoreType.DMA,
                pltpu.SemaphoreType.DMA,
                pltpu.SemaphoreType.DMA,
            ],
        )

    return make(DAUG), make(D)


_R = 1000


def _dense1_body(p_ref, x_ref, wl_ref, wr_ref, b_ref, o_ref):
    p = p_ref[0] + p_ref[1]
    s = jnp.sum(p[:, D:], axis=1)
    inv = 1.0 / jnp.maximum(s * (1.0 / DEGW), 1.0)
    mean = p[:, :D] * inv[:, None]
    z = (jnp.dot(mean, wl_ref[...], preferred_element_type=jnp.float32)
         + jnp.dot(x_ref[...], wr_ref[...], preferred_element_type=jnp.float32)
         + b_ref[...])
    o_ref[...] = jnp.maximum(z, 0.0)


def _dense2_body(p_ref, deg_ref, h_ref, wl_ref, wr_ref, b_ref, o_ref):
    s = jnp.sum(deg_ref[0] + deg_ref[1], axis=1)
    inv = 1.0 / jnp.maximum(s * (1.0 / DEGW), 1.0)
    mean = (p_ref[0] + p_ref[1]) * inv[:, None]
    z = (jnp.dot(mean, wl_ref[...], preferred_element_type=jnp.float32)
         + jnp.dot(h_ref[...], wr_ref[...], preferred_element_type=jnp.float32)
         + b_ref[...])
    m = jnp.max(z, axis=1, keepdims=True)
    lse = jnp.log(jnp.sum(jnp.exp(z - m), axis=1, keepdims=True)) + m
    o_ref[...] = z - lse


_dense1 = pl.pallas_call(
    _dense1_body,
    grid=(N // _R,),
    in_specs=[
        pl.BlockSpec((NC, _R, DAUG), lambda i: (0, i, 0)),
        pl.BlockSpec((_R, D), lambda i: (i, 0)),
        pl.BlockSpec((D, D), lambda i: (0, 0)),
        pl.BlockSpec((D, D), lambda i: (0, 0)),
        pl.BlockSpec((1, D), lambda i: (0, 0)),
    ],
    out_specs=pl.BlockSpec((_R, D), lambda i: (i, 0)),
    out_shape=jax.ShapeDtypeStruct((N, D), jnp.float32),
)

_dense2 = pl.pallas_call(
    _dense2_body,
    grid=(N // _R,),
    in_specs=[
        pl.BlockSpec((NC, _R, D), lambda i: (0, i, 0)),
        pl.BlockSpec((NC, _R, DEGW), lambda i: (0, i, 0)),
        pl.BlockSpec((_R, D), lambda i: (i, 0)),
        pl.BlockSpec((D, D), lambda i: (0, 0)),
        pl.BlockSpec((D, D), lambda i: (0, 0)),
        pl.BlockSpec((1, D), lambda i: (0, 0)),
    ],
    out_specs=pl.BlockSpec((_R, D), lambda i: (i, 0)),
    out_shape=jax.ShapeDtypeStruct((N, D), jnp.float32),
)


def kernel(x, edge_index, W1_l, b1_l, W1_r, W2_l, b2_l, W2_r):
    src = edge_index[0].astype(jnp.int32)
    dst = edge_index[1].astype(jnp.int32)
    nit = NFULL // 2
    sw = src.reshape(NW, EPW)
    dw = dst.reshape(NW, EPW)
    sblk = sw[:, :nit * 2 * CHUNK].reshape(NW, nit, 2, CHUNK)
    dblk = dw[:, :nit * 2 * CHUNK].reshape(NW, nit, 2, CHUNK)
    pk = jnp.concatenate([sblk, dblk], axis=2).reshape(NW, nit, 4 * CHUNK)
    rem = jnp.concatenate([sw[:, nit * 2 * CHUNK:],
                           dw[:, nit * 2 * CHUNK:]], axis=1)
    x_aug = jnp.concatenate([x, jnp.ones((N, DEGW), jnp.float32)], axis=1)
    z144 = jnp.zeros((CHUNK, DAUG), jnp.float32)
    z128 = jnp.zeros((CHUNK, D), jnp.float32)

    agg_aug, agg = _build_sc_kernels()
    p1 = agg_aug(x_aug, pk, rem, z144)
    h = _dense1(p1, x, W1_l.T, W1_r.T, b1_l.reshape(1, D))
    p2 = agg(h, pk, rem, z128)
    degc = lax.slice_in_dim(p1, D, DAUG, axis=2)
    out = _dense2(p2, degc, h, W2_l.T, W2_r.T, b2_l.reshape(1, D))
    return out


# speedup vs baseline: 9.1217x; 1.1001x over previous
import functools

import jax
import jax.numpy as jnp
from jax import lax
from jax.experimental import pallas as pl
from jax.experimental.pallas import tpu as pltpu
from jax.experimental.pallas import tpu_sc as plsc

N = 10000
E = 320000
D = 128
DEGW = 16
DAUG = D + DEGW
NC, NS = 2, 16
NW = NC * NS
EPW = E // NW
CHUNK = 128
NFULL = EPW // CHUNK
REM = EPW - NFULL * CHUNK
RPSA = 624
NTAIL = N - NS * RPSA
NBASE = NS * RPSA


def _sc_agg_body(width, *refs):
    (x_hbm, pk_hbm, rem_hbm, z_hbm,
     out_hbm,
     idxbuf, idxbuf2, idx_r, idx16, idxz, idxz112,
     rows, rows_b, rows16,
     acc, sem, sem_b, sem_s, sem_sb) = refs

    cid = lax.axis_index("c")
    sid = lax.axis_index("s")
    wid = cid * NS + sid
    r0 = pl.multiple_of(sid * RPSA, 16)
    _OFFS = ((0, 128), (128, 128), (256, 128), (384, 128), (512, 112))

    def fill_idx(ref, base, n):
        lanes = lax.iota(jnp.int32, 16)
        for j in range(0, n, 16):
            ref[pl.ds(j, 16)] = lanes + (base + j)

    pltpu.sync_copy(z_hbm, rows)
    for o, s in _OFFS:
        iref = idxz if s == 128 else idxz112
        fill_idx(iref, r0 + o, s)
        pltpu.sync_copy(rows.at[pl.ds(0, s)], acc.at[iref])

    @pl.when(sid == NS - 1)
    def _zero_tail():
        fill_idx(idx16, NBASE, NTAIL)
        pltpu.sync_copy(rows.at[pl.ds(0, NTAIL)], acc.at[idx16])

    plsc.subcore_barrier()


    def srcs(ib, half):
        return ib.at[pl.ds(half * CHUNK, CHUNK)]

    def dsts(ib, half):
        return ib.at[pl.ds((2 + half) * CHUNK, CHUNK)]

    NIT = NFULL // 2
    NPAIR = (NIT - 1) // 2

    pltpu.sync_copy(pk_hbm.at[wid, 0], idxbuf)
    pltpu.sync_copy(pk_hbm.at[wid, 1], idxbuf2)
    pltpu.async_copy(x_hbm.at[srcs(idxbuf, 0)], rows, sem)
    pltpu.async_copy(x_hbm.at[srcs(idxbuf, 1)], rows_b, sem_b)

    def pair_body(j, carry):
        e = 2 * j
        pltpu.make_async_copy(x_hbm.at[srcs(idxbuf, 0)], rows, sem).wait()
        sa = pltpu.async_copy(rows, acc.at[dsts(idxbuf, 0)], sem_s, add=True)
        pltpu.make_async_copy(x_hbm.at[srcs(idxbuf, 1)], rows_b, sem_b).wait()
        sb = pltpu.async_copy(rows_b, acc.at[dsts(idxbuf, 1)], sem_sb,
                              add=True)
        sa.wait()
        pltpu.async_copy(x_hbm.at[srcs(idxbuf2, 0)], rows, sem)
        sb.wait()
        pltpu.async_copy(x_hbm.at[srcs(idxbuf2, 1)], rows_b, sem_b)
        pltpu.sync_copy(pk_hbm.at[wid, e + 2], idxbuf)
        pltpu.make_async_copy(x_hbm.at[srcs(idxbuf2, 0)], rows, sem).wait()
        sa = pltpu.async_copy(rows, acc.at[dsts(idxbuf2, 0)], sem_s, add=True)
        pltpu.make_async_copy(x_hbm.at[srcs(idxbuf2, 1)], rows_b,
                              sem_b).wait()
        sb = pltpu.async_copy(rows_b, acc.at[dsts(idxbuf2, 1)], sem_sb,
                              add=True)
        sa.wait()
        pltpu.async_copy(x_hbm.at[srcs(idxbuf, 0)], rows, sem)
        sb.wait()
        pltpu.async_copy(x_hbm.at[srcs(idxbuf, 1)], rows_b, sem_b)

        @pl.when(j < NPAIR - 1)
        def _load_next_odd():
            pltpu.sync_copy(pk_hbm.at[wid, e + 3], idxbuf2)

        return carry

    lax.fori_loop(0, NPAIR, pair_body, 0)

    pltpu.make_async_copy(x_hbm.at[srcs(idxbuf, 0)], rows, sem).wait()
    sa = pltpu.async_copy(rows, acc.at[dsts(idxbuf, 0)], sem_s, add=True)
    pltpu.make_async_copy(x_hbm.at[srcs(idxbuf, 1)], rows_b, sem_b).wait()
    sb = pltpu.async_copy(rows_b, acc.at[dsts(idxbuf, 1)], sem_sb, add=True)
    sa.wait()
    sb.wait()

    pltpu.sync_copy(rem_hbm.at[wid], idx_r)
    pltpu.async_copy(x_hbm.at[idx_r.at[pl.ds(0, REM)]], rows16, sem).wait()
    pltpu.sync_copy(rows16, acc.at[idx_r.at[pl.ds(REM, REM)]], add=True)

    plsc.subcore_barrier()

    for o, s in _OFFS:
        iref = idxz if s == 128 else idxz112
        fill_idx(iref, r0 + o, s)
        pltpu.async_copy(acc.at[iref], rows.at[pl.ds(0, s)], sem).wait()
        pltpu.sync_copy(rows.at[pl.ds(0, s)],
                        out_hbm.at[cid, pl.ds(r0 + o, s)])

    @pl.when(sid == NS - 1)
    def _wb_tail():
        fill_idx(idx16, NBASE, NTAIL)
        pltpu.async_copy(acc.at[idx16], rows16, sem).wait()
        pltpu.sync_copy(rows16, out_hbm.at[cid, pl.ds(NBASE, NTAIL)])


@functools.lru_cache(maxsize=None)
def _build_sc_kernels():
    mesh = plsc.VectorSubcoreMesh(
        core_axis_name="c", subcore_axis_name="s",
        num_cores=NC, num_subcores=NS)

    def make(width):
        return pl.kernel(
            functools.partial(_sc_agg_body, width),
            out_type=jax.ShapeDtypeStruct((NC, N, width), jnp.float32),
            mesh=mesh,
            compiler_params=pltpu.CompilerParams(use_tc_tiling_on_sc=False),
            scratch_types=[
                pltpu.VMEM((4 * CHUNK,), jnp.int32),
                pltpu.VMEM((4 * CHUNK,), jnp.int32),
                pltpu.VMEM((2 * REM,), jnp.int32),
                pltpu.VMEM((REM,), jnp.int32),
                pltpu.VMEM((CHUNK,), jnp.int32),
                pltpu.VMEM((112,), jnp.int32),
                pltpu.VMEM((CHUNK, width), jnp.float32),
                pltpu.VMEM((CHUNK, width), jnp.float32),
                pltpu.VMEM((REM, width), jnp.float32),
                pltpu.VMEM_SHARED((N, width), jnp.float32),
                pltpu.SemaphoreType.DMA,
                pltpu.SemaphoreType.DMA,
                pltpu.SemaphoreType.DMA,
                pltpu.SemaphoreType.DMA,
            ],
        )

    return make(DAUG), make(D)


_R = 1000


def _dense1_body(p_ref, x_ref, wl_ref, wr_ref, b_ref, o_ref):
    p = p_ref[0] + p_ref[1]
    s = jnp.sum(p[:, D:], axis=1)
    inv = 1.0 / jnp.maximum(s * (1.0 / DEGW), 1.0)
    mean = p[:, :D] * inv[:, None]
    z = (jnp.dot(mean, wl_ref[...], preferred_element_type=jnp.float32)
         + jnp.dot(x_ref[...], wr_ref[...], preferred_element_type=jnp.float32)
         + b_ref[...])
    o_ref[...] = jnp.maximum(z, 0.0)


def _dense2_body(p_ref, deg_ref, h_ref, wl_ref, wr_ref, b_ref, o_ref):
    s = jnp.sum(deg_ref[0] + deg_ref[1], axis=1)
    inv = 1.0 / jnp.maximum(s * (1.0 / DEGW), 1.0)
    mean = (p_ref[0] + p_ref[1]) * inv[:, None]
    z = (jnp.dot(mean, wl_ref[...], preferred_element_type=jnp.float32)
         + jnp.dot(h_ref[...], wr_ref[...], preferred_element_type=jnp.float32)
         + b_ref[...])
    m = jnp.max(z, axis=1, keepdims=True)
    lse = jnp.log(jnp.sum(jnp.exp(z - m), axis=1, keepdims=True)) + m
    o_ref[...] = z - lse


_dense1 = pl.pallas_call(
    _dense1_body,
    grid=(N // _R,),
    in_specs=[
        pl.BlockSpec((NC, _R, DAUG), lambda i: (0, i, 0)),
        pl.BlockSpec((_R, D), lambda i: (i, 0)),
        pl.BlockSpec((D, D), lambda i: (0, 0)),
        pl.BlockSpec((D, D), lambda i: (0, 0)),
        pl.BlockSpec((1, D), lambda i: (0, 0)),
    ],
    out_specs=pl.BlockSpec((_R, D), lambda i: (i, 0)),
    out_shape=jax.ShapeDtypeStruct((N, D), jnp.float32),
)

_dense2 = pl.pallas_call(
    _dense2_body,
    grid=(N // _R,),
    in_specs=[
        pl.BlockSpec((NC, _R, D), lambda i: (0, i, 0)),
        pl.BlockSpec((NC, _R, DEGW), lambda i: (0, i, 0)),
        pl.BlockSpec((_R, D), lambda i: (i, 0)),
        pl.BlockSpec((D, D), lambda i: (0, 0)),
        pl.BlockSpec((D, D), lambda i: (0, 0)),
        pl.BlockSpec((1, D), lambda i: (0, 0)),
    ],
    out_specs=pl.BlockSpec((_R, D), lambda i: (i, 0)),
    out_shape=jax.ShapeDtypeStruct((N, D), jnp.float32),
)


def kernel(x, edge_index, W1_l, b1_l, W1_r, W2_l, b2_l, W2_r):
    src = edge_index[0].astype(jnp.int32)
    dst = edge_index[1].astype(jnp.int32)
    nit = NFULL // 2
    sw = src.reshape(NW, EPW)
    dw = dst.reshape(NW, EPW)
    sblk = sw[:, :nit * 2 * CHUNK].reshape(NW, nit, 2, CHUNK)
    dblk = dw[:, :nit * 2 * CHUNK].reshape(NW, nit, 2, CHUNK)
    pk = jnp.concatenate([sblk, dblk], axis=2).reshape(NW, nit, 4 * CHUNK)
    rem = jnp.concatenate([sw[:, nit * 2 * CHUNK:],
                           dw[:, nit * 2 * CHUNK:]], axis=1)
    x_aug = jnp.concatenate([x, jnp.ones((N, DEGW), jnp.float32)], axis=1)
    z144 = jnp.zeros((CHUNK, DAUG), jnp.float32)
    z128 = jnp.zeros((CHUNK, D), jnp.float32)

    agg_aug, agg = _build_sc_kernels()
    p1 = agg_aug(x_aug, pk, rem, z144)
    h = _dense1(p1, x, W1_l.T, W1_r.T, b1_l.reshape(1, D))
    p2 = agg(h, pk, rem, z128)
    degc = lax.slice_in_dim(p1, D, DAUG, axis=2)
    out = _dense2(p2, degc, h, W2_l.T, W2_r.T, b2_l.reshape(1, D))
    return out


# speedup vs baseline: 10.8704x; 1.1917x over previous
import functools

import jax
import jax.numpy as jnp
from jax import lax
from jax.experimental import pallas as pl
from jax.experimental.pallas import tpu as pltpu
from jax.experimental.pallas import tpu_sc as plsc

N = 10000
E = 320000
D = 128
DEGW = 16
NC, NS = 2, 16
NW = NC * NS
EPW = E // NW
CHUNK = 128
NFULL = EPW // CHUNK
REM = EPW - NFULL * CHUNK
RPSA = 624
NTAIL = N - NS * RPSA
NBASE = NS * RPSA


def _sc_agg_body(with_deg, *refs):
    if with_deg:
        (x_hbm, pk_hbm, rem_hbm, z_hbm, z16_hbm, ones_hbm,
         out_hbm, deg_hbm,
         idxbuf, idxbuf2, idx_r, idx16, idxz, idxz112,
         rows, rows_b, rows16, ones_v, t16,
         acc, degacc, sem, sem_b, sem_s, sem_sb, sem_d) = refs
    else:
        (x_hbm, pk_hbm, rem_hbm, z_hbm,
         out_hbm,
         idxbuf, idxbuf2, idx_r, idx16, idxz, idxz112,
         rows, rows_b, rows16,
         acc, sem, sem_b, sem_s, sem_sb) = refs
        degacc = ones_v = t16 = deg_hbm = None

    cid = lax.axis_index("c")
    sid = lax.axis_index("s")
    wid = cid * NS + sid
    r0 = pl.multiple_of(sid * RPSA, 16)
    _OFFS = ((0, 128), (128, 128), (256, 128), (384, 128), (512, 112))

    def fill_idx(ref, base, n):
        lanes = lax.iota(jnp.int32, 16)
        for j in range(0, n, 16):
            ref[pl.ds(j, 16)] = lanes + (base + j)

    pltpu.sync_copy(z_hbm, rows)
    if with_deg:
        pltpu.sync_copy(z16_hbm, ones_v)
    for o, s in _OFFS:
        iref = idxz if s == 128 else idxz112
        fill_idx(iref, r0 + o, s)
        pltpu.sync_copy(rows.at[pl.ds(0, s)], acc.at[iref])
        if with_deg:
            pltpu.sync_copy(ones_v.at[pl.ds(0, s)], degacc.at[iref])

    @pl.when(sid == NS - 1)
    def _zero_tail():
        fill_idx(idx16, NBASE, NTAIL)
        pltpu.sync_copy(rows.at[pl.ds(0, NTAIL)], acc.at[idx16])
        if with_deg:
            pltpu.sync_copy(ones_v.at[pl.ds(0, NTAIL)], degacc.at[idx16])

    if with_deg:
        pltpu.sync_copy(ones_hbm, ones_v)
    plsc.subcore_barrier()


    def srcs(ib, half):
        return ib.at[pl.ds(half * CHUNK, CHUNK)]

    def dsts(ib, half):
        return ib.at[pl.ds((2 + half) * CHUNK, CHUNK)]

    def half_step(ib, buf, g_sem, s_sem, half):
        pltpu.make_async_copy(x_hbm.at[srcs(ib, half)], buf, g_sem).wait()
        sc = pltpu.async_copy(buf, acc.at[dsts(ib, half)], s_sem, add=True)
        if with_deg:
            pltpu.async_copy(ones_v, degacc.at[dsts(ib, half)], sem_d,
                             add=True)
        return sc

    def deg_drain(ib):
        if with_deg:
            pltpu.make_async_copy(ones_v, degacc.at[dsts(ib, 0)],
                                  sem_d).wait()
            pltpu.make_async_copy(ones_v, degacc.at[dsts(ib, 1)],
                                  sem_d).wait()

    NIT = NFULL // 2
    NPAIR = (NIT - 1) // 2

    pltpu.sync_copy(pk_hbm.at[wid, 0], idxbuf)
    pltpu.sync_copy(pk_hbm.at[wid, 1], idxbuf2)
    pltpu.async_copy(x_hbm.at[srcs(idxbuf, 0)], rows, sem)
    pltpu.async_copy(x_hbm.at[srcs(idxbuf, 1)], rows_b, sem_b)

    def pair_body(j, carry):
        e = 2 * j
        sa = half_step(idxbuf, rows, sem, sem_s, 0)
        sb = half_step(idxbuf, rows_b, sem_b, sem_sb, 1)
        sa.wait()
        pltpu.async_copy(x_hbm.at[srcs(idxbuf2, 0)], rows, sem)
        sb.wait()
        pltpu.async_copy(x_hbm.at[srcs(idxbuf2, 1)], rows_b, sem_b)
        deg_drain(idxbuf)
        pltpu.sync_copy(pk_hbm.at[wid, e + 2], idxbuf)
        sa = half_step(idxbuf2, rows, sem, sem_s, 0)
        sb = half_step(idxbuf2, rows_b, sem_b, sem_sb, 1)
        sa.wait()
        pltpu.async_copy(x_hbm.at[srcs(idxbuf, 0)], rows, sem)
        sb.wait()
        pltpu.async_copy(x_hbm.at[srcs(idxbuf, 1)], rows_b, sem_b)
        deg_drain(idxbuf2)

        @pl.when(j < NPAIR - 1)
        def _load_next_odd():
            pltpu.sync_copy(pk_hbm.at[wid, e + 3], idxbuf2)

        return carry

    lax.fori_loop(0, NPAIR, pair_body, 0)

    sa = half_step(idxbuf, rows, sem, sem_s, 0)
    sb = half_step(idxbuf, rows_b, sem_b, sem_sb, 1)
    sa.wait()
    sb.wait()
    deg_drain(idxbuf)

    pltpu.sync_copy(rem_hbm.at[wid], idx_r)
    pltpu.async_copy(x_hbm.at[idx_r.at[pl.ds(0, REM)]], rows16, sem).wait()
    pltpu.sync_copy(rows16, acc.at[idx_r.at[pl.ds(REM, REM)]], add=True)
    if with_deg:
        pltpu.sync_copy(ones_v.at[pl.ds(0, REM)],
                        degacc.at[idx_r.at[pl.ds(REM, REM)]], add=True)

    plsc.subcore_barrier()

    for o, s in _OFFS:
        iref = idxz if s == 128 else idxz112
        fill_idx(iref, r0 + o, s)
        pltpu.async_copy(acc.at[iref], rows.at[pl.ds(0, s)], sem).wait()
        pltpu.sync_copy(rows.at[pl.ds(0, s)],
                        out_hbm.at[cid, pl.ds(r0 + o, s)])
        if with_deg:
            pltpu.async_copy(degacc.at[iref], ones_v.at[pl.ds(0, s)],
                             sem).wait()
            pltpu.sync_copy(ones_v.at[pl.ds(0, s)],
                            deg_hbm.at[cid, pl.ds(r0 + o, s)])

    @pl.when(sid == NS - 1)
    def _wb_tail():
        fill_idx(idx16, NBASE, NTAIL)
        pltpu.async_copy(acc.at[idx16], rows16, sem).wait()
        pltpu.sync_copy(rows16, out_hbm.at[cid, pl.ds(NBASE, NTAIL)])
        if with_deg:
            pltpu.async_copy(degacc.at[idx16], t16, sem).wait()
            pltpu.sync_copy(t16, deg_hbm.at[cid, pl.ds(NBASE, NTAIL)])


@functools.lru_cache(maxsize=None)
def _build_sc_kernels():
    mesh = plsc.VectorSubcoreMesh(
        core_axis_name="c", subcore_axis_name="s",
        num_cores=NC, num_subcores=NS)

    def idx_scratch():
        return [
            pltpu.VMEM((4 * CHUNK,), jnp.int32),
            pltpu.VMEM((4 * CHUNK,), jnp.int32),
            pltpu.VMEM((2 * REM,), jnp.int32),
            pltpu.VMEM((REM,), jnp.int32),
            pltpu.VMEM((CHUNK,), jnp.int32),
            pltpu.VMEM((112,), jnp.int32),
            pltpu.VMEM((CHUNK, D), jnp.float32),
            pltpu.VMEM((CHUNK, D), jnp.float32),
            pltpu.VMEM((REM, D), jnp.float32),
        ]

    agg_deg = pl.kernel(
        functools.partial(_sc_agg_body, True),
        out_type=(
            jax.ShapeDtypeStruct((NC, N, D), jnp.float32),
            jax.ShapeDtypeStruct((NC, N, DEGW), jnp.float32),
        ),
        mesh=mesh,
        compiler_params=pltpu.CompilerParams(use_tc_tiling_on_sc=False),
        scratch_types=idx_scratch() + [
            pltpu.VMEM((CHUNK, DEGW), jnp.float32),
            pltpu.VMEM((NTAIL, DEGW), jnp.float32),
            pltpu.VMEM_SHARED((N, D), jnp.float32),
            pltpu.VMEM_SHARED((N, DEGW), jnp.float32),
            pltpu.SemaphoreType.DMA,
            pltpu.SemaphoreType.DMA,
            pltpu.SemaphoreType.DMA,
            pltpu.SemaphoreType.DMA,
            pltpu.SemaphoreType.DMA,
        ],
    )
    agg = pl.kernel(
        functools.partial(_sc_agg_body, False),
        out_type=jax.ShapeDtypeStruct((NC, N, D), jnp.float32),
        mesh=mesh,
        compiler_params=pltpu.CompilerParams(use_tc_tiling_on_sc=False),
        scratch_types=idx_scratch() + [
            pltpu.VMEM_SHARED((N, D), jnp.float32),
            pltpu.SemaphoreType.DMA,
            pltpu.SemaphoreType.DMA,
            pltpu.SemaphoreType.DMA,
            pltpu.SemaphoreType.DMA,
        ],
    )
    return agg_deg, agg


_R = 1000


def _dense_body(last, p_ref, deg_ref, x_ref, wl_ref, wr_ref, b_ref, o_ref):
    s = jnp.sum(deg_ref[0] + deg_ref[1], axis=1)
    inv = 1.0 / jnp.maximum(s * (1.0 / DEGW), 1.0)
    mean = (p_ref[0] + p_ref[1]) * inv[:, None]
    z = (jnp.dot(mean, wl_ref[...], preferred_element_type=jnp.float32)
         + jnp.dot(x_ref[...], wr_ref[...], preferred_element_type=jnp.float32)
         + b_ref[...])
    if last:
        m = jnp.max(z, axis=1, keepdims=True)
        lse = jnp.log(jnp.sum(jnp.exp(z - m), axis=1, keepdims=True)) + m
        o_ref[...] = z - lse
    else:
        o_ref[...] = jnp.maximum(z, 0.0)


def _make_dense(last):
    return pl.pallas_call(
        functools.partial(_dense_body, last),
        grid=(N // _R,),
        in_specs=[
            pl.BlockSpec((NC, _R, D), lambda i: (0, i, 0)),
            pl.BlockSpec((NC, _R, DEGW), lambda i: (0, i, 0)),
            pl.BlockSpec((_R, D), lambda i: (i, 0)),
            pl.BlockSpec((D, D), lambda i: (0, 0)),
            pl.BlockSpec((D, D), lambda i: (0, 0)),
            pl.BlockSpec((1, D), lambda i: (0, 0)),
        ],
        out_specs=pl.BlockSpec((_R, D), lambda i: (i, 0)),
        out_shape=jax.ShapeDtypeStruct((N, D), jnp.float32),
    )


_dense1 = _make_dense(False)
_dense2 = _make_dense(True)


def kernel(x, edge_index, W1_l, b1_l, W1_r, W2_l, b2_l, W2_r):
    src = edge_index[0].astype(jnp.int32)
    dst = edge_index[1].astype(jnp.int32)
    nit = NFULL // 2
    sw = src.reshape(NW, EPW)
    dw = dst.reshape(NW, EPW)
    sblk = sw[:, :nit * 2 * CHUNK].reshape(NW, nit, 2, CHUNK)
    dblk = dw[:, :nit * 2 * CHUNK].reshape(NW, nit, 2, CHUNK)
    pk = jnp.concatenate([sblk, dblk], axis=2).reshape(NW, nit, 4 * CHUNK)
    rem = jnp.concatenate([sw[:, nit * 2 * CHUNK:],
                           dw[:, nit * 2 * CHUNK:]], axis=1)
    z128 = jnp.zeros((CHUNK, D), jnp.float32)
    z16 = jnp.zeros((CHUNK, DEGW), jnp.float32)
    ones16 = jnp.ones((CHUNK, DEGW), jnp.float32)

    agg_deg, agg = _build_sc_kernels()
    p1, deg2 = agg_deg(x, pk, rem, z128, z16, ones16)
    h = _dense1(p1, deg2, x, W1_l.T, W1_r.T, b1_l.reshape(1, D))
    p2 = agg(h, pk, rem, z128)
    out = _dense2(p2, deg2, h, W2_l.T, W2_r.T, b2_l.reshape(1, D))
    return out


# speedup vs baseline: 10.9482x; 1.0072x over previous
import functools

import jax
import jax.numpy as jnp
from jax import lax
from jax.experimental import pallas as pl
from jax.experimental.pallas import tpu as pltpu
from jax.experimental.pallas import tpu_sc as plsc

N = 10000
E = 320000
D = 128
DEGW = 16
NC, NS = 2, 16
NW = NC * NS
EPW = E // NW
CHUNK = 128
NFULL = EPW // CHUNK
REM = EPW - NFULL * CHUNK
RPSA = 624
NTAIL = N - NS * RPSA
NBASE = NS * RPSA


def _sc_agg_body(with_deg, *refs):
    if with_deg:
        (x_hbm, pk_hbm, rem_hbm, z_hbm, z16_hbm, ones_hbm,
         out_hbm, deg_hbm,
         idxbuf, idxbuf2, idx_r, idx16, idxz, idxz112,
         rows, rows_b, rows16, ones_v, t16,
         acc, degacc, sem, sem_b, sem_s, sem_sb, sem_d, sem_i) = refs
    else:
        (x_hbm, pk_hbm, rem_hbm, z_hbm,
         out_hbm,
         idxbuf, idxbuf2, idx_r, idx16, idxz, idxz112,
         rows, rows_b, rows16,
         acc, sem, sem_b, sem_s, sem_sb, sem_i) = refs
        degacc = ones_v = t16 = deg_hbm = None

    cid = lax.axis_index("c")
    sid = lax.axis_index("s")
    wid = cid * NS + sid
    r0 = pl.multiple_of(sid * RPSA, 16)
    _OFFS = ((0, 128), (128, 128), (256, 128), (384, 128), (512, 112))

    def fill_idx(ref, base, n):
        lanes = lax.iota(jnp.int32, 16)
        for j in range(0, n, 16):
            ref[pl.ds(j, 16)] = lanes + (base + j)

    pltpu.sync_copy(z_hbm, rows)
    if with_deg:
        pltpu.sync_copy(z16_hbm, ones_v)
    for o, s in _OFFS:
        iref = idxz if s == 128 else idxz112
        fill_idx(iref, r0 + o, s)
        pltpu.sync_copy(rows.at[pl.ds(0, s)], acc.at[iref])
        if with_deg:
            pltpu.sync_copy(ones_v.at[pl.ds(0, s)], degacc.at[iref])

    @pl.when(sid == NS - 1)
    def _zero_tail():
        fill_idx(idx16, NBASE, NTAIL)
        pltpu.sync_copy(rows.at[pl.ds(0, NTAIL)], acc.at[idx16])
        if with_deg:
            pltpu.sync_copy(ones_v.at[pl.ds(0, NTAIL)], degacc.at[idx16])

    if with_deg:
        pltpu.sync_copy(ones_hbm, ones_v)
    pltpu.sync_copy(pk_hbm.at[wid, 0], idxbuf)
    pltpu.sync_copy(pk_hbm.at[wid, 1], idxbuf2)
    pltpu.async_copy(x_hbm.at[idxbuf.at[pl.ds(0, CHUNK)]], rows, sem)
    pltpu.async_copy(x_hbm.at[idxbuf.at[pl.ds(CHUNK, CHUNK)]], rows_b, sem_b)
    plsc.subcore_barrier()


    def srcs(ib, half):
        return ib.at[pl.ds(half * CHUNK, CHUNK)]

    def dsts(ib, half):
        return ib.at[pl.ds((2 + half) * CHUNK, CHUNK)]

    def half_step(ib, buf, g_sem, s_sem, half):
        pltpu.make_async_copy(x_hbm.at[srcs(ib, half)], buf, g_sem).wait()
        sc = pltpu.async_copy(buf, acc.at[dsts(ib, half)], s_sem, add=True)
        if with_deg:
            pltpu.async_copy(ones_v, degacc.at[dsts(ib, half)], sem_d,
                             add=True)
        return sc

    def deg_drain(ib):
        if with_deg:
            pltpu.make_async_copy(ones_v, degacc.at[dsts(ib, 0)],
                                  sem_d).wait()
            pltpu.make_async_copy(ones_v, degacc.at[dsts(ib, 1)],
                                  sem_d).wait()

    NIT = NFULL // 2
    NPAIR = (NIT - 1) // 2

    def pair_body(j, carry):
        e = 2 * j
        sa = half_step(idxbuf, rows, sem, sem_s, 0)
        sb = half_step(idxbuf, rows_b, sem_b, sem_sb, 1)
        sa.wait()
        pltpu.async_copy(x_hbm.at[srcs(idxbuf2, 0)], rows, sem)
        sb.wait()
        pltpu.async_copy(x_hbm.at[srcs(idxbuf2, 1)], rows_b, sem_b)
        deg_drain(idxbuf)
        ld = pltpu.async_copy(pk_hbm.at[wid, e + 2], idxbuf, sem_i)
        sa = half_step(idxbuf2, rows, sem, sem_s, 0)
        sb = half_step(idxbuf2, rows_b, sem_b, sem_sb, 1)
        ld.wait()
        sa.wait()
        pltpu.async_copy(x_hbm.at[srcs(idxbuf, 0)], rows, sem)
        sb.wait()
        pltpu.async_copy(x_hbm.at[srcs(idxbuf, 1)], rows_b, sem_b)
        deg_drain(idxbuf2)

        @pl.when(j < NPAIR - 1)
        def _load_next_odd():
            pltpu.async_copy(pk_hbm.at[wid, e + 3], idxbuf2, sem_i).wait()

        return carry

    lax.fori_loop(0, NPAIR, pair_body, 0)

    sa = half_step(idxbuf, rows, sem, sem_s, 0)
    sb = half_step(idxbuf, rows_b, sem_b, sem_sb, 1)
    sa.wait()
    sb.wait()
    deg_drain(idxbuf)

    pltpu.sync_copy(rem_hbm.at[wid], idx_r)
    pltpu.async_copy(x_hbm.at[idx_r.at[pl.ds(0, REM)]], rows16, sem).wait()
    pltpu.sync_copy(rows16, acc.at[idx_r.at[pl.ds(REM, REM)]], add=True)
    if with_deg:
        pltpu.sync_copy(ones_v.at[pl.ds(0, REM)],
                        degacc.at[idx_r.at[pl.ds(REM, REM)]], add=True)

    plsc.subcore_barrier()

    bufs = (rows, rows_b)
    gsems = (sem, sem_b)
    wsems = (sem_s, sem_sb)
    irefs = (idxz, idxbuf.at[pl.ds(0, CHUNK)])
    gathers = [None] * len(_OFFS)
    writes = [None] * len(_OFFS)
    for k, (o, s) in enumerate(_OFFS):
        p = k % 2
        iref = irefs[p] if s == 128 else idxz112
        if k >= 2 and writes[k - 2] is not None:
            writes[k - 2].wait()
        fill_idx(iref, r0 + o, s)
        gathers[k] = pltpu.async_copy(
            acc.at[iref], bufs[p].at[pl.ds(0, s)], gsems[p])
        if k >= 1:
            gathers[k - 1].wait()
            o1, s1 = _OFFS[k - 1]
            writes[k - 1] = pltpu.async_copy(
                bufs[(k - 1) % 2].at[pl.ds(0, s1)],
                out_hbm.at[cid, pl.ds(r0 + o1, s1)], wsems[(k - 1) % 2])
    gathers[-1].wait()
    o1, s1 = _OFFS[-1]
    writes[-1] = pltpu.async_copy(
        bufs[(len(_OFFS) - 1) % 2].at[pl.ds(0, s1)],
        out_hbm.at[cid, pl.ds(r0 + o1, s1)], wsems[(len(_OFFS) - 1) % 2])
    writes[-2].wait()
    writes[-1].wait()
    if with_deg:
        for o, s in _OFFS:
            iref = idxz if s == 128 else idxz112
            fill_idx(iref, r0 + o, s)
            pltpu.async_copy(degacc.at[iref], ones_v.at[pl.ds(0, s)],
                             sem).wait()
            pltpu.sync_copy(ones_v.at[pl.ds(0, s)],
                            deg_hbm.at[cid, pl.ds(r0 + o, s)])

    @pl.when(sid == NS - 1)
    def _wb_tail():
        fill_idx(idx16, NBASE, NTAIL)
        pltpu.async_copy(acc.at[idx16], rows16, sem).wait()
        pltpu.sync_copy(rows16, out_hbm.at[cid, pl.ds(NBASE, NTAIL)])
        if with_deg:
            pltpu.async_copy(degacc.at[idx16], t16, sem).wait()
            pltpu.sync_copy(t16, deg_hbm.at[cid, pl.ds(NBASE, NTAIL)])


@functools.lru_cache(maxsize=None)
def _build_sc_kernels():
    mesh = plsc.VectorSubcoreMesh(
        core_axis_name="c", subcore_axis_name="s",
        num_cores=NC, num_subcores=NS)

    def idx_scratch():
        return [
            pltpu.VMEM((4 * CHUNK,), jnp.int32),
            pltpu.VMEM((4 * CHUNK,), jnp.int32),
            pltpu.VMEM((2 * REM,), jnp.int32),
            pltpu.VMEM((REM,), jnp.int32),
            pltpu.VMEM((CHUNK,), jnp.int32),
            pltpu.VMEM((112,), jnp.int32),
            pltpu.VMEM((CHUNK, D), jnp.float32),
            pltpu.VMEM((CHUNK, D), jnp.float32),
            pltpu.VMEM((REM, D), jnp.float32),
        ]

    agg_deg = pl.kernel(
        functools.partial(_sc_agg_body, True),
        out_type=(
            jax.ShapeDtypeStruct((NC, N, D), jnp.float32),
            jax.ShapeDtypeStruct((NC, N, DEGW), jnp.float32),
        ),
        mesh=mesh,
        compiler_params=pltpu.CompilerParams(use_tc_tiling_on_sc=False),
        scratch_types=idx_scratch() + [
            pltpu.VMEM((CHUNK, DEGW), jnp.float32),
            pltpu.VMEM((NTAIL, DEGW), jnp.float32),
            pltpu.VMEM_SHARED((N, D), jnp.float32),
            pltpu.VMEM_SHARED((N, DEGW), jnp.float32),
            pltpu.SemaphoreType.DMA,
            pltpu.SemaphoreType.DMA,
            pltpu.SemaphoreType.DMA,
            pltpu.SemaphoreType.DMA,
            pltpu.SemaphoreType.DMA,
            pltpu.SemaphoreType.DMA,
        ],
    )
    agg = pl.kernel(
        functools.partial(_sc_agg_body, False),
        out_type=jax.ShapeDtypeStruct((NC, N, D), jnp.float32),
        mesh=mesh,
        compiler_params=pltpu.CompilerParams(use_tc_tiling_on_sc=False),
        scratch_types=idx_scratch() + [
            pltpu.VMEM_SHARED((N, D), jnp.float32),
            pltpu.SemaphoreType.DMA,
            pltpu.SemaphoreType.DMA,
            pltpu.SemaphoreType.DMA,
            pltpu.SemaphoreType.DMA,
            pltpu.SemaphoreType.DMA,
        ],
    )
    return agg_deg, agg


_R = 1000


def _dense_body(last, p_ref, deg_ref, x_ref, wl_ref, wr_ref, b_ref, o_ref):
    s = jnp.sum(deg_ref[0] + deg_ref[1], axis=1)
    inv = 1.0 / jnp.maximum(s * (1.0 / DEGW), 1.0)
    mean = (p_ref[0] + p_ref[1]) * inv[:, None]
    z = (jnp.dot(mean, wl_ref[...], preferred_element_type=jnp.float32)
         + jnp.dot(x_ref[...], wr_ref[...], preferred_element_type=jnp.float32)
         + b_ref[...])
    if last:
        m = jnp.max(z, axis=1, keepdims=True)
        lse = jnp.log(jnp.sum(jnp.exp(z - m), axis=1, keepdims=True)) + m
        o_ref[...] = z - lse
    else:
        o_ref[...] = jnp.maximum(z, 0.0)


def _make_dense(last):
    return pl.pallas_call(
        functools.partial(_dense_body, last),
        grid=(N // _R,),
        in_specs=[
            pl.BlockSpec((NC, _R, D), lambda i: (0, i, 0)),
            pl.BlockSpec((NC, _R, DEGW), lambda i: (0, i, 0)),
            pl.BlockSpec((_R, D), lambda i: (i, 0)),
            pl.BlockSpec((D, D), lambda i: (0, 0)),
            pl.BlockSpec((D, D), lambda i: (0, 0)),
            pl.BlockSpec((1, D), lambda i: (0, 0)),
        ],
        out_specs=pl.BlockSpec((_R, D), lambda i: (i, 0)),
        out_shape=jax.ShapeDtypeStruct((N, D), jnp.float32),
    )


_dense1 = _make_dense(False)
_dense2 = _make_dense(True)


def kernel(x, edge_index, W1_l, b1_l, W1_r, W2_l, b2_l, W2_r):
    src = edge_index[0].astype(jnp.int32)
    dst = edge_index[1].astype(jnp.int32)
    nit = NFULL // 2
    sw = src.reshape(NW, EPW)
    dw = dst.reshape(NW, EPW)
    sblk = sw[:, :nit * 2 * CHUNK].reshape(NW, nit, 2, CHUNK)
    dblk = dw[:, :nit * 2 * CHUNK].reshape(NW, nit, 2, CHUNK)
    pk = jnp.concatenate([sblk, dblk], axis=2).reshape(NW, nit, 4 * CHUNK)
    rem = jnp.concatenate([sw[:, nit * 2 * CHUNK:],
                           dw[:, nit * 2 * CHUNK:]], axis=1)
    z128 = jnp.zeros((CHUNK, D), jnp.float32)
    z16 = jnp.zeros((CHUNK, DEGW), jnp.float32)
    ones16 = jnp.ones((CHUNK, DEGW), jnp.float32)

    agg_deg, agg = _build_sc_kernels()
    p1, deg2 = agg_deg(x, pk, rem, z128, z16, ones16)
    h = _dense1(p1, deg2, x, W1_l.T, W1_r.T, b1_l.reshape(1, D))
    p2 = agg(h, pk, rem, z128)
    out = _dense2(p2, deg2, h, W2_l.T, W2_r.T, b2_l.reshape(1, D))
    return out


# speedup vs baseline: 11.0492x; 1.0092x over previous
import functools

import jax
import jax.numpy as jnp
from jax import lax
from jax.experimental import pallas as pl
from jax.experimental.pallas import tpu as pltpu
from jax.experimental.pallas import tpu_sc as plsc

N = 10000
E = 320000
D = 128
DEGW = 16
NC, NS = 2, 16
NW = NC * NS
EPW = E // NW
CHUNK = 128
NFULL = EPW // CHUNK
REM = EPW - NFULL * CHUNK
RPSA = 624
NTAIL = N - NS * RPSA
NBASE = NS * RPSA


def _sc_agg_body(with_deg, *refs):
    if with_deg:
        (x_hbm, src_hbm, dst_hbm, z_hbm, z16_hbm, ones_hbm,
         out_hbm, deg_hbm,
         idxbuf, idxbuf2, idx_r, idx16, idxz, idxz112,
         rows, rows_b, rows16, ones_v, t16,
         acc, degacc, sem, sem_b, sem_s, sem_sb, sem_d, sem_i) = refs
    else:
        (x_hbm, src_hbm, dst_hbm, z_hbm,
         out_hbm,
         idxbuf, idxbuf2, idx_r, idx16, idxz, idxz112,
         rows, rows_b, rows16,
         acc, sem, sem_b, sem_s, sem_sb, sem_i) = refs
        degacc = ones_v = t16 = deg_hbm = None

    cid = lax.axis_index("c")
    sid = lax.axis_index("s")
    wid = cid * NS + sid
    r0 = pl.multiple_of(sid * RPSA, 16)
    _OFFS = ((0, 128), (128, 128), (256, 128), (384, 128), (512, 112))

    def fill_idx(ref, base, n):
        lanes = lax.iota(jnp.int32, 16)
        for j in range(0, n, 16):
            ref[pl.ds(j, 16)] = lanes + (base + j)

    def load_idx_sync(ib, k):
        pltpu.sync_copy(src_hbm.at[wid, pl.ds(k * 2 * CHUNK, 2 * CHUNK)],
                        ib.at[pl.ds(0, 2 * CHUNK)])
        pltpu.sync_copy(dst_hbm.at[wid, pl.ds(k * 2 * CHUNK, 2 * CHUNK)],
                        ib.at[pl.ds(2 * CHUNK, 2 * CHUNK)])

    def load_idx_async(ib, k):
        pltpu.async_copy(src_hbm.at[wid, pl.ds(k * 2 * CHUNK, 2 * CHUNK)],
                         ib.at[pl.ds(0, 2 * CHUNK)], sem_i)
        return pltpu.async_copy(
            dst_hbm.at[wid, pl.ds(k * 2 * CHUNK, 2 * CHUNK)],
            ib.at[pl.ds(2 * CHUNK, 2 * CHUNK)], sem_i)

    def idx_wait(ib, k):
        pltpu.make_async_copy(
            src_hbm.at[wid, pl.ds(k * 2 * CHUNK, 2 * CHUNK)],
            ib.at[pl.ds(0, 2 * CHUNK)], sem_i).wait()
        pltpu.make_async_copy(
            dst_hbm.at[wid, pl.ds(k * 2 * CHUNK, 2 * CHUNK)],
            ib.at[pl.ds(2 * CHUNK, 2 * CHUNK)], sem_i).wait()

    pltpu.sync_copy(z_hbm, rows)
    if with_deg:
        pltpu.sync_copy(z16_hbm, ones_v)
    for o, s in _OFFS:
        iref = idxz if s == 128 else idxz112
        fill_idx(iref, r0 + o, s)
        pltpu.sync_copy(rows.at[pl.ds(0, s)], acc.at[iref])
        if with_deg:
            pltpu.sync_copy(ones_v.at[pl.ds(0, s)], degacc.at[iref])

    @pl.when(sid == NS - 1)
    def _zero_tail():
        fill_idx(idx16, NBASE, NTAIL)
        pltpu.sync_copy(rows.at[pl.ds(0, NTAIL)], acc.at[idx16])
        if with_deg:
            pltpu.sync_copy(ones_v.at[pl.ds(0, NTAIL)], degacc.at[idx16])

    if with_deg:
        pltpu.sync_copy(ones_hbm, ones_v)
    load_idx_sync(idxbuf, 0)
    load_idx_sync(idxbuf2, 1)
    pltpu.async_copy(x_hbm.at[idxbuf.at[pl.ds(0, CHUNK)]], rows, sem)
    pltpu.async_copy(x_hbm.at[idxbuf.at[pl.ds(CHUNK, CHUNK)]], rows_b, sem_b)
    plsc.subcore_barrier()


    def srcs(ib, half):
        return ib.at[pl.ds(half * CHUNK, CHUNK)]

    def dsts(ib, half):
        return ib.at[pl.ds((2 + half) * CHUNK, CHUNK)]

    def half_step(ib, buf, g_sem, s_sem, half):
        pltpu.make_async_copy(x_hbm.at[srcs(ib, half)], buf, g_sem).wait()
        sc = pltpu.async_copy(buf, acc.at[dsts(ib, half)], s_sem, add=True)
        if with_deg:
            pltpu.async_copy(ones_v, degacc.at[dsts(ib, half)], sem_d,
                             add=True)
        return sc

    def deg_drain(ib):
        if with_deg:
            pltpu.make_async_copy(ones_v, degacc.at[dsts(ib, 0)],
                                  sem_d).wait()
            pltpu.make_async_copy(ones_v, degacc.at[dsts(ib, 1)],
                                  sem_d).wait()

    NIT = NFULL // 2
    NPAIR = (NIT - 1) // 2

    def pair_body(j, carry):
        e = 2 * j
        sa = half_step(idxbuf, rows, sem, sem_s, 0)
        sb = half_step(idxbuf, rows_b, sem_b, sem_sb, 1)
        sa.wait()
        pltpu.async_copy(x_hbm.at[srcs(idxbuf2, 0)], rows, sem)
        sb.wait()
        pltpu.async_copy(x_hbm.at[srcs(idxbuf2, 1)], rows_b, sem_b)
        deg_drain(idxbuf)
        load_idx_async(idxbuf, e + 2)
        sa = half_step(idxbuf2, rows, sem, sem_s, 0)
        sb = half_step(idxbuf2, rows_b, sem_b, sem_sb, 1)
        idx_wait(idxbuf, e + 2)
        sa.wait()
        pltpu.async_copy(x_hbm.at[srcs(idxbuf, 0)], rows, sem)
        sb.wait()
        pltpu.async_copy(x_hbm.at[srcs(idxbuf, 1)], rows_b, sem_b)
        deg_drain(idxbuf2)

        @pl.when(j < NPAIR - 1)
        def _load_next_odd():
            load_idx_async(idxbuf2, e + 3)
            idx_wait(idxbuf2, e + 3)

        return carry

    lax.fori_loop(0, NPAIR, pair_body, 0)

    sa = half_step(idxbuf, rows, sem, sem_s, 0)
    sb = half_step(idxbuf, rows_b, sem_b, sem_sb, 1)
    sa.wait()
    sb.wait()
    deg_drain(idxbuf)

    pltpu.sync_copy(src_hbm.at[wid, pl.ds(NFULL * CHUNK, REM)],
                    idx_r.at[pl.ds(0, REM)])
    pltpu.sync_copy(dst_hbm.at[wid, pl.ds(NFULL * CHUNK, REM)],
                    idx_r.at[pl.ds(REM, REM)])
    pltpu.async_copy(x_hbm.at[idx_r.at[pl.ds(0, REM)]], rows16, sem).wait()
    pltpu.sync_copy(rows16, acc.at[idx_r.at[pl.ds(REM, REM)]], add=True)
    if with_deg:
        pltpu.sync_copy(ones_v.at[pl.ds(0, REM)],
                        degacc.at[idx_r.at[pl.ds(REM, REM)]], add=True)

    plsc.subcore_barrier()

    bufs = (rows, rows_b)
    gsems = (sem, sem_b)
    wsems = (sem_s, sem_sb)
    irefs = (idxz, idxbuf.at[pl.ds(0, CHUNK)])
    gathers = [None] * len(_OFFS)
    writes = [None] * len(_OFFS)
    for k, (o, s) in enumerate(_OFFS):
        p = k % 2
        iref = irefs[p] if s == 128 else idxz112
        if k >= 2 and writes[k - 2] is not None:
            writes[k - 2].wait()
        fill_idx(iref, r0 + o, s)
        gathers[k] = pltpu.async_copy(
            acc.at[iref], bufs[p].at[pl.ds(0, s)], gsems[p])
        if k >= 1:
            gathers[k - 1].wait()
            o1, s1 = _OFFS[k - 1]
            writes[k - 1] = pltpu.async_copy(
                bufs[(k - 1) % 2].at[pl.ds(0, s1)],
                out_hbm.at[cid, pl.ds(r0 + o1, s1)], wsems[(k - 1) % 2])
    gathers[-1].wait()
    o1, s1 = _OFFS[-1]
    writes[-1] = pltpu.async_copy(
        bufs[(len(_OFFS) - 1) % 2].at[pl.ds(0, s1)],
        out_hbm.at[cid, pl.ds(r0 + o1, s1)], wsems[(len(_OFFS) - 1) % 2])
    writes[-2].wait()
    writes[-1].wait()
    if with_deg:
        for o, s in _OFFS:
            iref = idxz if s == 128 else idxz112
            fill_idx(iref, r0 + o, s)
            pltpu.async_copy(degacc.at[iref], ones_v.at[pl.ds(0, s)],
                             sem).wait()
            pltpu.sync_copy(ones_v.at[pl.ds(0, s)],
                            deg_hbm.at[cid, pl.ds(r0 + o, s)])

    @pl.when(sid == NS - 1)
    def _wb_tail():
        fill_idx(idx16, NBASE, NTAIL)
        pltpu.async_copy(acc.at[idx16], rows16, sem).wait()
        pltpu.sync_copy(rows16, out_hbm.at[cid, pl.ds(NBASE, NTAIL)])
        if with_deg:
            pltpu.async_copy(degacc.at[idx16], t16, sem).wait()
            pltpu.sync_copy(t16, deg_hbm.at[cid, pl.ds(NBASE, NTAIL)])


@functools.lru_cache(maxsize=None)
def _build_sc_kernels():
    mesh = plsc.VectorSubcoreMesh(
        core_axis_name="c", subcore_axis_name="s",
        num_cores=NC, num_subcores=NS)

    def idx_scratch():
        return [
            pltpu.VMEM((4 * CHUNK,), jnp.int32),
            pltpu.VMEM((4 * CHUNK,), jnp.int32),
            pltpu.VMEM((2 * REM,), jnp.int32),
            pltpu.VMEM((REM,), jnp.int32),
            pltpu.VMEM((CHUNK,), jnp.int32),
            pltpu.VMEM((112,), jnp.int32),
            pltpu.VMEM((CHUNK, D), jnp.float32),
            pltpu.VMEM((CHUNK, D), jnp.float32),
            pltpu.VMEM((REM, D), jnp.float32),
        ]

    agg_deg = pl.kernel(
        functools.partial(_sc_agg_body, True),
        out_type=(
            jax.ShapeDtypeStruct((NC, N, D), jnp.float32),
            jax.ShapeDtypeStruct((NC, N, DEGW), jnp.float32),
        ),
        mesh=mesh,
        compiler_params=pltpu.CompilerParams(use_tc_tiling_on_sc=False),
        scratch_types=idx_scratch() + [
            pltpu.VMEM((CHUNK, DEGW), jnp.float32),
            pltpu.VMEM((NTAIL, DEGW), jnp.float32),
            pltpu.VMEM_SHARED((N, D), jnp.float32),
            pltpu.VMEM_SHARED((N, DEGW), jnp.float32),
            pltpu.SemaphoreType.DMA,
            pltpu.SemaphoreType.DMA,
            pltpu.SemaphoreType.DMA,
            pltpu.SemaphoreType.DMA,
            pltpu.SemaphoreType.DMA,
            pltpu.SemaphoreType.DMA,
        ],
    )
    agg = pl.kernel(
        functools.partial(_sc_agg_body, False),
        out_type=jax.ShapeDtypeStruct((NC, N, D), jnp.float32),
        mesh=mesh,
        compiler_params=pltpu.CompilerParams(use_tc_tiling_on_sc=False),
        scratch_types=idx_scratch() + [
            pltpu.VMEM_SHARED((N, D), jnp.float32),
            pltpu.SemaphoreType.DMA,
            pltpu.SemaphoreType.DMA,
            pltpu.SemaphoreType.DMA,
            pltpu.SemaphoreType.DMA,
            pltpu.SemaphoreType.DMA,
        ],
    )
    return agg_deg, agg


_R = 1000


def _dense_body(last, p_ref, deg_ref, x_ref, wl_ref, wr_ref, b_ref, o_ref):
    s = jnp.sum(deg_ref[0] + deg_ref[1], axis=1)
    inv = 1.0 / jnp.maximum(s * (1.0 / DEGW), 1.0)
    mean = (p_ref[0] + p_ref[1]) * inv[:, None]
    z = (jnp.dot(mean, wl_ref[...], preferred_element_type=jnp.float32)
         + jnp.dot(x_ref[...], wr_ref[...], preferred_element_type=jnp.float32)
         + b_ref[...])
    if last:
        m = jnp.max(z, axis=1, keepdims=True)
        lse = jnp.log(jnp.sum(jnp.exp(z - m), axis=1, keepdims=True)) + m
        o_ref[...] = z - lse
    else:
        o_ref[...] = jnp.maximum(z, 0.0)


def _make_dense(last):
    return pl.pallas_call(
        functools.partial(_dense_body, last),
        grid=(N // _R,),
        in_specs=[
            pl.BlockSpec((NC, _R, D), lambda i: (0, i, 0)),
            pl.BlockSpec((NC, _R, DEGW), lambda i: (0, i, 0)),
            pl.BlockSpec((_R, D), lambda i: (i, 0)),
            pl.BlockSpec((D, D), lambda i: (0, 0)),
            pl.BlockSpec((D, D), lambda i: (0, 0)),
            pl.BlockSpec((1, D), lambda i: (0, 0)),
        ],
        out_specs=pl.BlockSpec((_R, D), lambda i: (i, 0)),
        out_shape=jax.ShapeDtypeStruct((N, D), jnp.float32),
    )


_dense1 = _make_dense(False)
_dense2 = _make_dense(True)


def kernel(x, edge_index, W1_l, b1_l, W1_r, W2_l, b2_l, W2_r):
    src = edge_index[0].astype(jnp.int32)
    dst = edge_index[1].astype(jnp.int32)
    sw = src.reshape(NW, EPW)
    dw = dst.reshape(NW, EPW)
    z128 = jnp.zeros((CHUNK, D), jnp.float32)
    z16 = jnp.zeros((CHUNK, DEGW), jnp.float32)
    ones16 = jnp.ones((CHUNK, DEGW), jnp.float32)

    agg_deg, agg = _build_sc_kernels()
    p1, deg2 = agg_deg(x, sw, dw, z128, z16, ones16)
    h = _dense1(p1, deg2, x, W1_l.T, W1_r.T, b1_l.reshape(1, D))
    p2 = agg(h, sw, dw, z128)
    out = _dense2(p2, deg2, h, W2_l.T, W2_r.T, b2_l.reshape(1, D))
    return out
